# Initial kernel scaffold; baseline (speedup 1.0000x reference)
import jax
import jax.numpy as jnp
from jax.experimental import pallas as pl


def kernel(E_u_0, E_i_0, edge_index):
    raise NotImplementedError("write your pallas kernel here")


# speedup vs baseline: 3.9224x; 3.9224x over previous
import functools

import jax
import jax.numpy as jnp
from jax import lax
from jax.experimental import pallas as pl
from jax.experimental.pallas import tpu as pltpu
from jax.experimental.pallas import tpu_sc as plsc

NU = 5000
D = 128
NE = 320000
NC, NS, L = 2, 16, 16
NW = NC * NS
NT = 5120
TRASH = NU
ER = 2560
EPC = ER // NC
EPT = EPC // NS
RPW = NT // NW
RPS = NT // NS
DEGW = 128

_mesh = plsc.VectorSubcoreMesh(
    core_axis_name="c", subcore_axis_name="s", num_cores=NC, num_subcores=NS)

_f32 = jnp.float32
_i32 = jnp.int32


def _fill(ref, nrows, ncolblk, val, dtype=_f32):
    def body(r, carry):
        for cb in range(ncolblk):
            ref[r, pl.ds(cb * L, L)] = jnp.full((L,), val, dtype)
        return carry
    lax.fori_loop(0, nrows, body, 0)


def _rsqrt16(x):
    s = 0.5 * (x + 1.0)
    for _ in range(15):
        s = 0.5 * (s + x / s)
    return 1.0 / s


@functools.partial(
    pl.kernel,
    out_type=[
        jax.ShapeDtypeStruct((NC, NT, DEGW), _f32),
        jax.ShapeDtypeStruct((NC, NT, DEGW), _f32),
    ],
    mesh=_mesh,
    scratch_types=[
        pltpu.VMEM_SHARED((NT, DEGW), _f32),
        pltpu.VMEM_SHARED((NT, DEGW), _f32),
        pltpu.VMEM((EPT, 128), _i32),
        pltpu.VMEM((EPT, 128), _i32),
        pltpu.VMEM((128, DEGW), _f32),
    ],
)
def _deg_kernel(u2d, i2d, degu_o, degi_o, sh_du, sh_di, idx_u, idx_i, ones):
    c = lax.axis_index("c")
    s = lax.axis_index("s")
    _fill(ones, 128, DEGW // L, 0.0)
    for k0 in range(0, RPS, 80):
        pltpu.sync_copy(ones.at[pl.ds(0, 80)], sh_du.at[pl.ds(s * RPS + k0, 80)])
        pltpu.sync_copy(ones.at[pl.ds(0, 80)], sh_di.at[pl.ds(s * RPS + k0, 80)])
    _fill(ones, 128, DEGW // L, 1.0)
    plsc.subcore_barrier()

    r0 = c * EPC + s * EPT
    pltpu.sync_copy(u2d.at[pl.ds(r0, EPT)], idx_u)
    pltpu.sync_copy(i2d.at[pl.ds(r0, EPT)], idx_i)

    def sub_body(r, carry):
        for cb in range(8):
            idx_i[r, pl.ds(cb * L, L)] = idx_i[r, pl.ds(cb * L, L)] - NU
        return carry
    lax.fori_loop(0, EPT, sub_body, 0)

    def edge_body(j, carry):
        pltpu.sync_copy(ones, sh_du.at[idx_u.at[j]], add=True)
        pltpu.sync_copy(ones, sh_di.at[idx_i.at[j]], add=True)
        return carry
    lax.fori_loop(0, EPT, edge_body, 0)
    plsc.subcore_barrier()

    sl = pl.ds(s * RPS, RPS)
    pltpu.sync_copy(sh_du.at[sl], degu_o.at[c, sl])
    pltpu.sync_copy(sh_di.at[sl], degi_o.at[c, sl])


@functools.partial(
    pl.kernel,
    out_type=[
        jax.ShapeDtypeStruct((NT, DEGW), _f32),
        jax.ShapeDtypeStruct((NT, DEGW), _f32),
        jax.ShapeDtypeStruct((NT, D), _f32),
        jax.ShapeDtypeStruct((NT, D), _f32),
    ],
    mesh=_mesh,
    scratch_types=[
        pltpu.VMEM((RPW, D), _f32),
        pltpu.VMEM((RPW, D), _f32),
        pltpu.VMEM((RPW, DEGW), _f32),
        pltpu.VMEM((RPW, DEGW), _f32),
        pltpu.VMEM((RPW, DEGW), _f32),
    ],
)
def _scale_kernel(degu_p, degi_p, eu0, ei0, ru_o, ri_o, au_o, ai_o,
                  e_buf, a_buf, d0, d1, r_buf):
    c = lax.axis_index("c")
    s = lax.axis_index("s")
    w = s * NC + c
    sl = pl.ds(w * RPW, RPW)
    for deg_p, e_in, r_out, a_out in (
        (degu_p, eu0, ru_o, au_o),
        (degi_p, ei0, ri_o, ai_o),
    ):
        pltpu.sync_copy(deg_p.at[0, sl], d0)
        pltpu.sync_copy(deg_p.at[1, sl], d1)
        pltpu.sync_copy(e_in.at[sl], e_buf)

        def row_body(r, carry):
            d = d0[r, pl.ds(0, L)] + d1[r, pl.ds(0, L)]
            y = jnp.where(d > 0.0, _rsqrt16(d), 0.0)
            r_buf[r, pl.ds(0, L)] = y
            for cb in range(8):
                a_buf[r, pl.ds(cb * L, L)] = y * e_buf[r, pl.ds(cb * L, L)]
            return carry
        lax.fori_loop(0, RPW, row_body, 0)
        pltpu.sync_copy(r_buf, r_out.at[sl])
        pltpu.sync_copy(a_buf, a_out.at[sl])


@functools.partial(
    pl.kernel,
    out_type=[
        jax.ShapeDtypeStruct((NC, NT, D), _f32),
        jax.ShapeDtypeStruct((NC, NT, D), _f32),
    ],
    mesh=_mesh,
    scratch_types=[
        pltpu.VMEM_SHARED((NT, D), _f32),
        pltpu.VMEM_SHARED((NT, D), _f32),
        pltpu.VMEM((EPT, 128), _i32),
        pltpu.VMEM((EPT, 128), _i32),
        pltpu.VMEM((128, D), _f32),
        pltpu.SemaphoreType.DMA,
    ],
)
def _edge_kernel(u2d, i2d, au, ai, tu_o, ti_o, sh_tu, sh_ti,
                 idx_u, idx_i, rows, sem1):
    c = lax.axis_index("c")
    s = lax.axis_index("s")
    _fill(rows, 128, D // L, 0.0)
    for k0 in range(0, RPS, 80):
        pltpu.sync_copy(rows.at[pl.ds(0, 80)], sh_tu.at[pl.ds(s * RPS + k0, 80)])
        pltpu.sync_copy(rows.at[pl.ds(0, 80)], sh_ti.at[pl.ds(s * RPS + k0, 80)])
    plsc.subcore_barrier()

    r0 = c * EPC + s * EPT
    pltpu.sync_copy(u2d.at[pl.ds(r0, EPT)], idx_u)
    pltpu.sync_copy(i2d.at[pl.ds(r0, EPT)], idx_i)

    def sub_body(r, carry):
        for cb in range(8):
            idx_i[r, pl.ds(cb * L, L)] = idx_i[r, pl.ds(cb * L, L)] - NU
        return carry
    lax.fori_loop(0, EPT, sub_body, 0)

    def edge_body(j, carry):
        pltpu.async_copy(ai.at[idx_i.at[j]], rows, sem1).wait()
        pltpu.sync_copy(rows, sh_tu.at[idx_u.at[j]], add=True)
        pltpu.async_copy(au.at[idx_u.at[j]], rows, sem1).wait()
        pltpu.sync_copy(rows, sh_ti.at[idx_i.at[j]], add=True)
        return carry
    lax.fori_loop(0, EPT, edge_body, 0)
    plsc.subcore_barrier()

    sl = pl.ds(s * RPS, RPS)
    pltpu.sync_copy(sh_tu.at[sl], tu_o.at[c, sl])
    pltpu.sync_copy(sh_ti.at[sl], ti_o.at[c, sl])


@functools.partial(
    pl.kernel,
    out_type=[
        jax.ShapeDtypeStruct((NT, D), _f32),
        jax.ShapeDtypeStruct((NT, D), _f32),
        jax.ShapeDtypeStruct((NT, D), _f32),
        jax.ShapeDtypeStruct((NT, D), _f32),
    ],
    mesh=_mesh,
    scratch_types=[
        pltpu.VMEM((RPW, D), _f32),
        pltpu.VMEM((RPW, D), _f32),
        pltpu.VMEM((RPW, D), _f32),
        pltpu.VMEM((RPW, DEGW), _f32),
    ],
)
def _mid_kernel(ru_i, ri_i, tu_p, ti_p, eu0, ei0, au_o, ai_o, pu_o, pi_o,
                t0, t1, e_buf, r_buf):
    c = lax.axis_index("c")
    s = lax.axis_index("s")
    w = s * NC + c
    sl = pl.ds(w * RPW, RPW)
    for r_in, t_p, e_in, a_out, p_out in (
        (ru_i, tu_p, eu0, au_o, pu_o),
        (ri_i, ti_p, ei0, ai_o, pi_o),
    ):
        pltpu.sync_copy(t_p.at[0, sl], t0)
        pltpu.sync_copy(t_p.at[1, sl], t1)
        pltpu.sync_copy(e_in.at[sl], e_buf)
        pltpu.sync_copy(r_in.at[sl], r_buf)

        def row_body(r, carry):
            y = r_buf[r, pl.ds(0, L)]
            for cb in range(8):
                cs = pl.ds(cb * L, L)
                t = t0[r, cs] + t1[r, cs]
                t0[r, cs] = y * y * t
                e_buf[r, cs] = e_buf[r, cs] + y * t
            return carry
        lax.fori_loop(0, RPW, row_body, 0)
        pltpu.sync_copy(t0, a_out.at[sl])
        pltpu.sync_copy(e_buf, p_out.at[sl])


@functools.partial(
    pl.kernel,
    out_type=[
        jax.ShapeDtypeStruct((NT, D), _f32),
        jax.ShapeDtypeStruct((NT, D), _f32),
    ],
    mesh=_mesh,
    scratch_types=[
        pltpu.VMEM((RPW, D), _f32),
        pltpu.VMEM((RPW, D), _f32),
        pltpu.VMEM((RPW, D), _f32),
        pltpu.VMEM((RPW, DEGW), _f32),
    ],
)
def _final_kernel(ru_i, ri_i, tu_p, ti_p, pu_i, pi_i, su_o, si_o,
                  t0, t1, p_buf, r_buf):
    c = lax.axis_index("c")
    s = lax.axis_index("s")
    w = s * NC + c
    sl = pl.ds(w * RPW, RPW)
    for r_in, t_p, p_in, s_out in (
        (ru_i, tu_p, pu_i, su_o),
        (ri_i, ti_p, pi_i, si_o),
    ):
        pltpu.sync_copy(t_p.at[0, sl], t0)
        pltpu.sync_copy(t_p.at[1, sl], t1)
        pltpu.sync_copy(p_in.at[sl], p_buf)
        pltpu.sync_copy(r_in.at[sl], r_buf)

        def row_body(r, carry):
            y = r_buf[r, pl.ds(0, L)]
            for cb in range(8):
                cs = pl.ds(cb * L, L)
                p_buf[r, cs] = p_buf[r, cs] + y * (t0[r, cs] + t1[r, cs])
            return carry
        lax.fori_loop(0, RPW, row_body, 0)
        pltpu.sync_copy(p_buf, s_out.at[sl])


def kernel(E_u_0, E_i_0, edge_index):
    u = edge_index[:, 0].astype(_i32)
    it = edge_index[:, 1].astype(_i32)
    pad = ER * 128 - NE
    u2d = jnp.concatenate([u, jnp.full((pad,), TRASH, _i32)]).reshape(ER, 128)
    i2d = jnp.concatenate([it, jnp.full((pad,), TRASH + NU, _i32)]).reshape(ER, 128)
    eu0 = jnp.pad(E_u_0, ((0, NT - NU), (0, 0)))
    ei0 = jnp.pad(E_i_0, ((0, NT - NU), (0, 0)))

    degu_p, degi_p = _deg_kernel(u2d, i2d)
    ru, ri, au1, ai1 = _scale_kernel(degu_p, degi_p, eu0, ei0)
    tu1, ti1 = _edge_kernel(u2d, i2d, au1, ai1)
    au2, ai2, pu, pi = _mid_kernel(ru, ri, tu1, ti1, eu0, ei0)
    tu2, ti2 = _edge_kernel(u2d, i2d, au2, ai2)
    su, si = _final_kernel(ru, ri, tu2, ti2, pu, pi)
    return su[:NU], si[:NU]


# speedup vs baseline: 10.2129x; 2.6037x over previous
import functools

import jax
import jax.numpy as jnp
from jax import lax
from jax.experimental import pallas as pl
from jax.experimental.pallas import tpu as pltpu
from jax.experimental.pallas import tpu_sc as plsc

NU = 5000
D = 128
NE = 320000
NC, NS, L = 2, 16, 16
NW = NC * NS
NT = 5120
TRASH = NU
ER = 2560
EPC = ER // NC
EPT = EPC // NS
RPW = NT // NW
RPS = NT // NS
DEGW = 128

_mesh = plsc.VectorSubcoreMesh(
    core_axis_name="c", subcore_axis_name="s", num_cores=NC, num_subcores=NS)

_f32 = jnp.float32
_i32 = jnp.int32


def _fill(ref, nrows, ncolblk, val, dtype=_f32):
    def body(r, carry):
        for cb in range(ncolblk):
            ref[r, pl.ds(cb * L, L)] = jnp.full((L,), val, dtype)
        return carry
    lax.fori_loop(0, nrows, body, 0)


def _rsqrt16(x):
    s = 0.5 * (x + 1.0)
    for _ in range(15):
        s = 0.5 * (s + x / s)
    return 1.0 / s


@functools.partial(
    pl.kernel,
    out_type=[
        jax.ShapeDtypeStruct((NC, NT, DEGW), _f32),
        jax.ShapeDtypeStruct((NC, NT, DEGW), _f32),
    ],
    mesh=_mesh,
    scratch_types=[
        pltpu.VMEM_SHARED((NT, DEGW), _f32),
        pltpu.VMEM_SHARED((NT, DEGW), _f32),
        pltpu.VMEM((EPT, 128), _i32),
        pltpu.VMEM((EPT, 128), _i32),
        pltpu.VMEM((128, DEGW), _f32),
    ],
)
def _deg_kernel(u2d, i2d, degu_o, degi_o, sh_du, sh_di, idx_u, idx_i, ones):
    c = lax.axis_index("c")
    s = lax.axis_index("s")
    _fill(ones, 128, DEGW // L, 0.0)
    for k0 in range(0, RPS, 80):
        pltpu.sync_copy(ones.at[pl.ds(0, 80)], sh_du.at[pl.ds(s * RPS + k0, 80)])
        pltpu.sync_copy(ones.at[pl.ds(0, 80)], sh_di.at[pl.ds(s * RPS + k0, 80)])
    _fill(ones, 128, DEGW // L, 1.0)
    plsc.subcore_barrier()

    r0 = c * EPC + s * EPT
    pltpu.sync_copy(u2d.at[pl.ds(r0, EPT)], idx_u)
    pltpu.sync_copy(i2d.at[pl.ds(r0, EPT)], idx_i)

    def sub_body(r, carry):
        for cb in range(8):
            idx_i[r, pl.ds(cb * L, L)] = idx_i[r, pl.ds(cb * L, L)] - NU
        return carry
    lax.fori_loop(0, EPT, sub_body, 0)

    def edge_body(j, carry):
        pltpu.sync_copy(ones, sh_du.at[idx_u.at[j]], add=True)
        pltpu.sync_copy(ones, sh_di.at[idx_i.at[j]], add=True)
        return carry
    lax.fori_loop(0, EPT, edge_body, 0)
    plsc.subcore_barrier()

    sl = pl.ds(s * RPS, RPS)
    pltpu.sync_copy(sh_du.at[sl], degu_o.at[c, sl])
    pltpu.sync_copy(sh_di.at[sl], degi_o.at[c, sl])


@functools.partial(
    pl.kernel,
    out_type=[
        jax.ShapeDtypeStruct((NT, DEGW), _f32),
        jax.ShapeDtypeStruct((NT, DEGW), _f32),
        jax.ShapeDtypeStruct((NT, D), _f32),
        jax.ShapeDtypeStruct((NT, D), _f32),
    ],
    mesh=_mesh,
    scratch_types=[
        pltpu.VMEM((RPW, D), _f32),
        pltpu.VMEM((RPW, D), _f32),
        pltpu.VMEM((RPW, DEGW), _f32),
        pltpu.VMEM((RPW, DEGW), _f32),
        pltpu.VMEM((RPW, DEGW), _f32),
    ],
)
def _scale_kernel(degu_p, degi_p, eu0, ei0, ru_o, ri_o, au_o, ai_o,
                  e_buf, a_buf, d0, d1, r_buf):
    c = lax.axis_index("c")
    s = lax.axis_index("s")
    w = s * NC + c
    sl = pl.ds(w * RPW, RPW)
    for deg_p, e_in, r_out, a_out in (
        (degu_p, eu0, ru_o, au_o),
        (degi_p, ei0, ri_o, ai_o),
    ):
        pltpu.sync_copy(deg_p.at[0, sl], d0)
        pltpu.sync_copy(deg_p.at[1, sl], d1)
        pltpu.sync_copy(e_in.at[sl], e_buf)

        def row_body(r, carry):
            d = d0[r, pl.ds(0, L)] + d1[r, pl.ds(0, L)]
            y = jnp.where(d > 0.0, _rsqrt16(d), 0.0)
            r_buf[r, pl.ds(0, L)] = y
            for cb in range(8):
                a_buf[r, pl.ds(cb * L, L)] = y * e_buf[r, pl.ds(cb * L, L)]
            return carry
        lax.fori_loop(0, RPW, row_body, 0)
        pltpu.sync_copy(r_buf, r_out.at[sl])
        pltpu.sync_copy(a_buf, a_out.at[sl])


@functools.partial(
    pl.kernel,
    out_type=[
        jax.ShapeDtypeStruct((NC, NT, D), _f32),
        jax.ShapeDtypeStruct((NC, NT, D), _f32),
    ],
    mesh=_mesh,
    scratch_types=[
        pltpu.VMEM_SHARED((NT, D), _f32),
        pltpu.VMEM_SHARED((NT, D), _f32),
        pltpu.VMEM((EPT, 128), _i32),
        pltpu.VMEM((EPT, 128), _i32),
        pltpu.VMEM((128, D), _f32),
        pltpu.SemaphoreType.DMA,
    ],
)
def _edge_kernel(u2d, i2d, au, ai, tu_o, ti_o, sh_tu, sh_ti,
                 idx_u, idx_i, rows, sem1):
    c = lax.axis_index("c")
    s = lax.axis_index("s")
    _fill(rows, 128, D // L, 0.0)
    for k0 in range(0, RPS, 80):
        pltpu.sync_copy(rows.at[pl.ds(0, 80)], sh_tu.at[pl.ds(s * RPS + k0, 80)])
        pltpu.sync_copy(rows.at[pl.ds(0, 80)], sh_ti.at[pl.ds(s * RPS + k0, 80)])
    plsc.subcore_barrier()

    r0 = c * EPC + s * EPT
    pltpu.sync_copy(u2d.at[pl.ds(r0, EPT)], idx_u)
    pltpu.sync_copy(i2d.at[pl.ds(r0, EPT)], idx_i)

    def sub_body(r, carry):
        for cb in range(8):
            idx_i[r, pl.ds(cb * L, L)] = idx_i[r, pl.ds(cb * L, L)] - NU
        return carry
    lax.fori_loop(0, EPT, sub_body, 0)

    def edge_body(j, carry):
        pltpu.async_copy(ai.at[idx_i.at[j]], rows, sem1).wait()
        pltpu.sync_copy(rows, sh_tu.at[idx_u.at[j]], add=True)
        pltpu.async_copy(au.at[idx_u.at[j]], rows, sem1).wait()
        pltpu.sync_copy(rows, sh_ti.at[idx_i.at[j]], add=True)
        return carry
    lax.fori_loop(0, EPT, edge_body, 0)
    plsc.subcore_barrier()

    sl = pl.ds(s * RPS, RPS)
    pltpu.sync_copy(sh_tu.at[sl], tu_o.at[c, sl])
    pltpu.sync_copy(sh_ti.at[sl], ti_o.at[c, sl])


@functools.partial(
    pl.kernel,
    out_type=[
        jax.ShapeDtypeStruct((NT, D), _f32),
        jax.ShapeDtypeStruct((NT, D), _f32),
        jax.ShapeDtypeStruct((NT, D), _f32),
        jax.ShapeDtypeStruct((NT, D), _f32),
    ],
    mesh=_mesh,
    scratch_types=[
        pltpu.VMEM((RPW, D), _f32),
        pltpu.VMEM((RPW, D), _f32),
        pltpu.VMEM((RPW, D), _f32),
        pltpu.VMEM((RPW, DEGW), _f32),
    ],
)
def _mid_kernel(ru_i, ri_i, tu_p, ti_p, eu0, ei0, au_o, ai_o, pu_o, pi_o,
                t0, t1, e_buf, r_buf):
    c = lax.axis_index("c")
    s = lax.axis_index("s")
    w = s * NC + c
    sl = pl.ds(w * RPW, RPW)
    for r_in, t_p, e_in, a_out, p_out in (
        (ru_i, tu_p, eu0, au_o, pu_o),
        (ri_i, ti_p, ei0, ai_o, pi_o),
    ):
        pltpu.sync_copy(t_p.at[0, sl], t0)
        pltpu.sync_copy(t_p.at[1, sl], t1)
        pltpu.sync_copy(e_in.at[sl], e_buf)
        pltpu.sync_copy(r_in.at[sl], r_buf)

        def row_body(r, carry):
            y = r_buf[r, pl.ds(0, L)]
            for cb in range(8):
                cs = pl.ds(cb * L, L)
                t = t0[r, cs] + t1[r, cs]
                t0[r, cs] = y * y * t
                e_buf[r, cs] = e_buf[r, cs] + y * t
            return carry
        lax.fori_loop(0, RPW, row_body, 0)
        pltpu.sync_copy(t0, a_out.at[sl])
        pltpu.sync_copy(e_buf, p_out.at[sl])


@functools.partial(
    pl.kernel,
    out_type=[
        jax.ShapeDtypeStruct((NT, D), _f32),
        jax.ShapeDtypeStruct((NT, D), _f32),
    ],
    mesh=_mesh,
    scratch_types=[
        pltpu.VMEM((RPW, D), _f32),
        pltpu.VMEM((RPW, D), _f32),
        pltpu.VMEM((RPW, D), _f32),
        pltpu.VMEM((RPW, DEGW), _f32),
    ],
)
def _final_kernel(ru_i, ri_i, tu_p, ti_p, pu_i, pi_i, su_o, si_o,
                  t0, t1, p_buf, r_buf):
    c = lax.axis_index("c")
    s = lax.axis_index("s")
    w = s * NC + c
    sl = pl.ds(w * RPW, RPW)
    for r_in, t_p, p_in, s_out in (
        (ru_i, tu_p, pu_i, su_o),
        (ri_i, ti_p, pi_i, si_o),
    ):
        pltpu.sync_copy(t_p.at[0, sl], t0)
        pltpu.sync_copy(t_p.at[1, sl], t1)
        pltpu.sync_copy(p_in.at[sl], p_buf)
        pltpu.sync_copy(r_in.at[sl], r_buf)

        def row_body(r, carry):
            y = r_buf[r, pl.ds(0, L)]
            for cb in range(8):
                cs = pl.ds(cb * L, L)
                p_buf[r, cs] = p_buf[r, cs] + y * (t0[r, cs] + t1[r, cs])
            return carry
        lax.fori_loop(0, RPW, row_body, 0)
        pltpu.sync_copy(p_buf, s_out.at[sl])


def kernel(E_u_0, E_i_0, edge_index):
    u = edge_index[:, 0].astype(_i32)
    it = edge_index[:, 1].astype(_i32)
    pad = ER * 128 - NE
    padv = TRASH + (jnp.arange(pad, dtype=_i32) % (NT - NU))
    u2d = jnp.concatenate([u, padv]).reshape(ER, 128)
    i2d = jnp.concatenate([it, padv + NU]).reshape(ER, 128)
    eu0 = jnp.pad(E_u_0, ((0, NT - NU), (0, 0)))
    ei0 = jnp.pad(E_i_0, ((0, NT - NU), (0, 0)))

    degu_p, degi_p = _deg_kernel(u2d, i2d)
    ru, ri, au1, ai1 = _scale_kernel(degu_p, degi_p, eu0, ei0)
    tu1, ti1 = _edge_kernel(u2d, i2d, au1, ai1)
    au2, ai2, pu, pi = _mid_kernel(ru, ri, tu1, ti1, eu0, ei0)
    tu2, ti2 = _edge_kernel(u2d, i2d, au2, ai2)
    su, si = _final_kernel(ru, ri, tu2, ti2, pu, pi)
    return su[:NU], si[:NU]


# speedup vs baseline: 12.6885x; 1.2424x over previous
import functools

import jax
import jax.numpy as jnp
from jax import lax
from jax.experimental import pallas as pl
from jax.experimental.pallas import tpu as pltpu
from jax.experimental.pallas import tpu_sc as plsc

NU = 5000
D = 128
NE = 320000
NC, NS, L = 2, 16, 16
NW = NC * NS
NT = 5120
TRASH = NU
ER = 2560
EPC = ER // NC
EPT = EPC // NS
CH = 16
NCHK = EPT // CH
RPW = NT // NW
RPS = NT // NS
DEGW = 128

_mesh = plsc.VectorSubcoreMesh(
    core_axis_name="c", subcore_axis_name="s", num_cores=NC, num_subcores=NS)

_f32 = jnp.float32
_i32 = jnp.int32


def _fill(ref, nrows, ncolblk, val, dtype=_f32):
    def body(r, carry):
        for cb in range(ncolblk):
            ref[r, pl.ds(cb * L, L)] = jnp.full((L,), val, dtype)
        return carry
    lax.fori_loop(0, nrows, body, 0)


def _rsqrt16(x):
    s = 0.5 * (x + 1.0)
    for _ in range(15):
        s = 0.5 * (s + x / s)
    return 1.0 / s


@functools.partial(
    pl.kernel,
    out_type=[
        jax.ShapeDtypeStruct((NC, NT, DEGW), _f32),
        jax.ShapeDtypeStruct((NC, NT, DEGW), _f32),
    ],
    mesh=_mesh,
    scratch_types=[
        pltpu.VMEM_SHARED((NT, DEGW), _f32),
        pltpu.VMEM_SHARED((NT, DEGW), _f32),
        pltpu.VMEM((EPT, 128), _i32),
        pltpu.VMEM((EPT, 128), _i32),
        pltpu.VMEM((128, DEGW), _f32),
    ],
)
def _deg_kernel(u2d, i2d, degu_o, degi_o, sh_du, sh_di, idx_u, idx_i, ones):
    c = lax.axis_index("c")
    s = lax.axis_index("s")
    _fill(ones, 128, DEGW // L, 0.0)
    for k0 in range(0, RPS, 80):
        pltpu.sync_copy(ones.at[pl.ds(0, 80)], sh_du.at[pl.ds(s * RPS + k0, 80)])
        pltpu.sync_copy(ones.at[pl.ds(0, 80)], sh_di.at[pl.ds(s * RPS + k0, 80)])
    _fill(ones, 128, DEGW // L, 1.0)
    plsc.subcore_barrier()

    r0 = c * EPC + s * EPT
    pltpu.sync_copy(u2d.at[pl.ds(r0, EPT)], idx_u)
    pltpu.sync_copy(i2d.at[pl.ds(r0, EPT)], idx_i)

    def edge_body(j, carry):
        pltpu.sync_copy(ones, sh_du.at[idx_u.at[j]], add=True)
        pltpu.sync_copy(ones, sh_di.at[idx_i.at[j]], add=True)
        return carry
    lax.fori_loop(0, EPT, edge_body, 0)
    plsc.subcore_barrier()

    sl = pl.ds(s * RPS, RPS)
    pltpu.sync_copy(sh_du.at[sl], degu_o.at[c, sl])
    pltpu.sync_copy(sh_di.at[sl], degi_o.at[c, sl])


@functools.partial(
    pl.kernel,
    out_type=[
        jax.ShapeDtypeStruct((NT, DEGW), _f32),
        jax.ShapeDtypeStruct((NT, DEGW), _f32),
        jax.ShapeDtypeStruct((NT, D), _f32),
        jax.ShapeDtypeStruct((NT, D), _f32),
    ],
    mesh=_mesh,
    scratch_types=[
        pltpu.VMEM((RPW, D), _f32),
        pltpu.VMEM((RPW, D), _f32),
        pltpu.VMEM((RPW, DEGW), _f32),
        pltpu.VMEM((RPW, DEGW), _f32),
        pltpu.VMEM((RPW, DEGW), _f32),
    ],
)
def _scale_kernel(degu_p, degi_p, eu0, ei0, ru_o, ri_o, au_o, ai_o,
                  e_buf, a_buf, d0, d1, r_buf):
    c = lax.axis_index("c")
    s = lax.axis_index("s")
    w = s * NC + c
    sl = pl.ds(w * RPW, RPW)
    for deg_p, e_in, r_out, a_out in (
        (degu_p, eu0, ru_o, au_o),
        (degi_p, ei0, ri_o, ai_o),
    ):
        pltpu.sync_copy(deg_p.at[0, sl], d0)
        pltpu.sync_copy(deg_p.at[1, sl], d1)
        pltpu.sync_copy(e_in.at[sl], e_buf)

        def row_body(r, carry):
            d = d0[r, pl.ds(0, L)] + d1[r, pl.ds(0, L)]
            y = jnp.where(d > 0.0, _rsqrt16(d), 0.0)
            r_buf[r, pl.ds(0, L)] = y
            for cb in range(8):
                a_buf[r, pl.ds(cb * L, L)] = y * e_buf[r, pl.ds(cb * L, L)]
            return carry
        lax.fori_loop(0, RPW, row_body, 0)
        pltpu.sync_copy(r_buf, r_out.at[sl])
        pltpu.sync_copy(a_buf, a_out.at[sl])


@functools.partial(
    pl.kernel,
    out_type=[
        jax.ShapeDtypeStruct((NC, NT, D), _f32),
        jax.ShapeDtypeStruct((NC, NT, D), _f32),
    ],
    mesh=_mesh,
    scratch_types=[
        pltpu.VMEM_SHARED((NT, D), _f32),
        pltpu.VMEM_SHARED((NT, D), _f32),
        pltpu.VMEM((2, CH, 128), _i32),
        pltpu.VMEM((2, CH, 128), _i32),
        pltpu.VMEM((128, D), _f32),
        pltpu.VMEM((128, D), _f32),
        pltpu.SemaphoreType.DMA,
        pltpu.SemaphoreType.DMA,
        pltpu.SemaphoreType.DMA,
        pltpu.SemaphoreType.DMA,
        pltpu.SemaphoreType.DMA,
    ],
)
def _edge_kernel(u2d, i2d, au, ai, tu_o, ti_o, sh_tu, sh_ti,
                 idx_u, idx_i, buf_a, buf_b, sg0, sg1, ss0, ss1, spf):
    c = lax.axis_index("c")
    s = lax.axis_index("s")
    _fill(buf_a, 128, D // L, 0.0)
    for k0 in range(0, RPS, 80):
        pltpu.sync_copy(buf_a.at[pl.ds(0, 80)], sh_tu.at[pl.ds(s * RPS + k0, 80)])
        pltpu.sync_copy(buf_a.at[pl.ds(0, 80)], sh_ti.at[pl.ds(s * RPS + k0, 80)])
    plsc.subcore_barrier()

    r0 = c * EPC + s * EPT
    pltpu.sync_copy(u2d.at[pl.ds(r0, CH)], idx_u.at[0])
    pltpu.sync_copy(i2d.at[pl.ds(r0, CH)], idx_i.at[0])

    for ch in range(NCHK):
        slot, nxt = ch % 2, (ch + 1) % 2
        if ch > 0:
            pltpu.make_async_copy(buf_a, sh_tu.at[idx_u.at[slot, 0]], ss0).wait()
            pltpu.make_async_copy(buf_b, sh_ti.at[idx_i.at[slot, 0]], ss1).wait()
        if ch + 1 < NCHK:
            r1 = r0 + (ch + 1) * CH
            pf_u = pltpu.async_copy(u2d.at[pl.ds(r1, CH)], idx_u.at[nxt], spf)
            pf_i = pltpu.async_copy(i2d.at[pl.ds(r1, CH)], idx_i.at[nxt], spf)

        def edge_body(jj, carry, slot=slot):
            iu = idx_u.at[slot, jj]
            ii = idx_i.at[slot, jj]

            @pl.when(jj > 0)
            def _():
                pltpu.make_async_copy(buf_a, sh_tu.at[iu], ss0).wait()
            pltpu.async_copy(ai.at[ii], buf_a, sg0).wait()
            pltpu.async_copy(buf_a, sh_tu.at[iu], ss0, add=True)

            @pl.when(jj > 0)
            def _():
                pltpu.make_async_copy(buf_b, sh_ti.at[ii], ss1).wait()
            pltpu.async_copy(au.at[iu], buf_b, sg1).wait()
            pltpu.async_copy(buf_b, sh_ti.at[ii], ss1, add=True)
            return carry
        lax.fori_loop(0, CH, edge_body, 0)
        if ch + 1 < NCHK:
            pf_u.wait()
            pf_i.wait()
    pltpu.make_async_copy(buf_a, sh_tu.at[idx_u.at[(NCHK - 1) % 2, 0]], ss0).wait()
    pltpu.make_async_copy(buf_b, sh_ti.at[idx_i.at[(NCHK - 1) % 2, 0]], ss1).wait()
    plsc.subcore_barrier()

    sl = pl.ds(s * RPS, RPS)
    pltpu.sync_copy(sh_tu.at[sl], tu_o.at[c, sl])
    pltpu.sync_copy(sh_ti.at[sl], ti_o.at[c, sl])


@functools.partial(
    pl.kernel,
    out_type=[
        jax.ShapeDtypeStruct((NT, D), _f32),
        jax.ShapeDtypeStruct((NT, D), _f32),
        jax.ShapeDtypeStruct((NT, D), _f32),
        jax.ShapeDtypeStruct((NT, D), _f32),
    ],
    mesh=_mesh,
    scratch_types=[
        pltpu.VMEM((RPW, D), _f32),
        pltpu.VMEM((RPW, D), _f32),
        pltpu.VMEM((RPW, D), _f32),
        pltpu.VMEM((RPW, DEGW), _f32),
    ],
)
def _mid_kernel(ru_i, ri_i, tu_p, ti_p, eu0, ei0, au_o, ai_o, pu_o, pi_o,
                t0, t1, e_buf, r_buf):
    c = lax.axis_index("c")
    s = lax.axis_index("s")
    w = s * NC + c
    sl = pl.ds(w * RPW, RPW)
    for r_in, t_p, e_in, a_out, p_out in (
        (ru_i, tu_p, eu0, au_o, pu_o),
        (ri_i, ti_p, ei0, ai_o, pi_o),
    ):
        pltpu.sync_copy(t_p.at[0, sl], t0)
        pltpu.sync_copy(t_p.at[1, sl], t1)
        pltpu.sync_copy(e_in.at[sl], e_buf)
        pltpu.sync_copy(r_in.at[sl], r_buf)

        def row_body(r, carry):
            y = r_buf[r, pl.ds(0, L)]
            for cb in range(8):
                cs = pl.ds(cb * L, L)
                t = t0[r, cs] + t1[r, cs]
                t0[r, cs] = y * y * t
                e_buf[r, cs] = e_buf[r, cs] + y * t
            return carry
        lax.fori_loop(0, RPW, row_body, 0)
        pltpu.sync_copy(t0, a_out.at[sl])
        pltpu.sync_copy(e_buf, p_out.at[sl])


@functools.partial(
    pl.kernel,
    out_type=[
        jax.ShapeDtypeStruct((NT, D), _f32),
        jax.ShapeDtypeStruct((NT, D), _f32),
    ],
    mesh=_mesh,
    scratch_types=[
        pltpu.VMEM((RPW, D), _f32),
        pltpu.VMEM((RPW, D), _f32),
        pltpu.VMEM((RPW, D), _f32),
        pltpu.VMEM((RPW, DEGW), _f32),
    ],
)
def _final_kernel(ru_i, ri_i, tu_p, ti_p, pu_i, pi_i, su_o, si_o,
                  t0, t1, p_buf, r_buf):
    c = lax.axis_index("c")
    s = lax.axis_index("s")
    w = s * NC + c
    sl = pl.ds(w * RPW, RPW)
    for r_in, t_p, p_in, s_out in (
        (ru_i, tu_p, pu_i, su_o),
        (ri_i, ti_p, pi_i, si_o),
    ):
        pltpu.sync_copy(t_p.at[0, sl], t0)
        pltpu.sync_copy(t_p.at[1, sl], t1)
        pltpu.sync_copy(p_in.at[sl], p_buf)
        pltpu.sync_copy(r_in.at[sl], r_buf)

        def row_body(r, carry):
            y = r_buf[r, pl.ds(0, L)]
            for cb in range(8):
                cs = pl.ds(cb * L, L)
                p_buf[r, cs] = p_buf[r, cs] + y * (t0[r, cs] + t1[r, cs])
            return carry
        lax.fori_loop(0, RPW, row_body, 0)
        pltpu.sync_copy(p_buf, s_out.at[sl])


def kernel(E_u_0, E_i_0, edge_index):
    u = edge_index[:, 0].astype(_i32)
    it = edge_index[:, 1].astype(_i32) - NU
    pad = ER * 128 - NE
    padv = TRASH + (jnp.arange(pad, dtype=_i32) % (NT - NU))
    u2d = jnp.concatenate([u, padv]).reshape(ER, 128)
    i2d = jnp.concatenate([it, padv]).reshape(ER, 128)
    eu0 = jnp.pad(E_u_0, ((0, NT - NU), (0, 0)))
    ei0 = jnp.pad(E_i_0, ((0, NT - NU), (0, 0)))

    degu_p, degi_p = _deg_kernel(u2d, i2d)
    ru, ri, au1, ai1 = _scale_kernel(degu_p, degi_p, eu0, ei0)
    tu1, ti1 = _edge_kernel(u2d, i2d, au1, ai1)
    au2, ai2, pu, pi = _mid_kernel(ru, ri, tu1, ti1, eu0, ei0)
    tu2, ti2 = _edge_kernel(u2d, i2d, au2, ai2)
    su, si = _final_kernel(ru, ri, tu2, ti2, pu, pi)
    return su[:NU], si[:NU]


# speedup vs baseline: 12.7560x; 1.0053x over previous
import functools

import jax
import jax.numpy as jnp
from jax import lax
from jax.experimental import pallas as pl
from jax.experimental.pallas import tpu as pltpu
from jax.experimental.pallas import tpu_sc as plsc

NU = 5000
D = 128
NE = 320000
NC, NS, L = 2, 16, 16
NW = NC * NS
NT = 5120
TRASH = NU
ER = 2560
EPC = ER // NC
EPT = EPC // NS
CH = 16
NCHK = EPT // CH
RPW = NT // NW
RPS = NT // NS
DEGW = 128

_mesh = plsc.VectorSubcoreMesh(
    core_axis_name="c", subcore_axis_name="s", num_cores=NC, num_subcores=NS)

_f32 = jnp.float32
_i32 = jnp.int32


def _fill(ref, nrows, ncolblk, val, dtype=_f32):
    def body(r, carry):
        for cb in range(ncolblk):
            ref[r, pl.ds(cb * L, L)] = jnp.full((L,), val, dtype)
        return carry
    lax.fori_loop(0, nrows, body, 0)


def _rsqrt16(x):
    s = 0.5 * (x + 1.0)
    for _ in range(15):
        s = 0.5 * (s + x / s)
    return 1.0 / s


@functools.partial(
    pl.kernel,
    out_type=[
        jax.ShapeDtypeStruct((NC, NT, DEGW), _f32),
        jax.ShapeDtypeStruct((NC, NT, DEGW), _f32),
    ],
    mesh=_mesh,
    scratch_types=[
        pltpu.VMEM_SHARED((NT, DEGW), _f32),
        pltpu.VMEM_SHARED((NT, DEGW), _f32),
        pltpu.VMEM((EPT, 128), _i32),
        pltpu.VMEM((EPT, 128), _i32),
        pltpu.VMEM((128, DEGW), _f32),
        pltpu.SemaphoreType.DMA,
        pltpu.SemaphoreType.DMA,
    ],
)
def _deg_kernel(u2d, i2d, degu_o, degi_o, sh_du, sh_di, idx_u, idx_i, ones,
                s0, s1):
    c = lax.axis_index("c")
    s = lax.axis_index("s")
    _fill(ones, 128, DEGW // L, 0.0)
    for k0 in range(0, RPS, 80):
        pltpu.sync_copy(ones.at[pl.ds(0, 80)], sh_du.at[pl.ds(s * RPS + k0, 80)])
        pltpu.sync_copy(ones.at[pl.ds(0, 80)], sh_di.at[pl.ds(s * RPS + k0, 80)])
    _fill(ones, 128, DEGW // L, 1.0)
    plsc.subcore_barrier()

    r0 = c * EPC + s * EPT
    pltpu.sync_copy(u2d.at[pl.ds(r0, EPT)], idx_u)
    pltpu.sync_copy(i2d.at[pl.ds(r0, EPT)], idx_i)

    def edge_body(j, carry):
        @pl.when(j > 0)
        def _():
            pltpu.make_async_copy(ones, sh_du.at[idx_u.at[j]], s0).wait()
            pltpu.make_async_copy(ones, sh_di.at[idx_i.at[j]], s1).wait()
        pltpu.async_copy(ones, sh_du.at[idx_u.at[j]], s0, add=True)
        pltpu.async_copy(ones, sh_di.at[idx_i.at[j]], s1, add=True)
        return carry
    lax.fori_loop(0, EPT, edge_body, 0)
    pltpu.make_async_copy(ones, sh_du.at[idx_u.at[0]], s0).wait()
    pltpu.make_async_copy(ones, sh_di.at[idx_i.at[0]], s1).wait()
    plsc.subcore_barrier()

    sl = pl.ds(s * RPS, RPS)
    pltpu.sync_copy(sh_du.at[sl], degu_o.at[c, sl])
    pltpu.sync_copy(sh_di.at[sl], degi_o.at[c, sl])


@functools.partial(
    pl.kernel,
    out_type=[
        jax.ShapeDtypeStruct((NT, DEGW), _f32),
        jax.ShapeDtypeStruct((NT, DEGW), _f32),
        jax.ShapeDtypeStruct((NT, D), _f32),
        jax.ShapeDtypeStruct((NT, D), _f32),
    ],
    mesh=_mesh,
    scratch_types=[
        pltpu.VMEM((RPW, D), _f32),
        pltpu.VMEM((RPW, D), _f32),
        pltpu.VMEM((RPW, DEGW), _f32),
        pltpu.VMEM((RPW, DEGW), _f32),
        pltpu.VMEM((RPW, DEGW), _f32),
    ],
)
def _scale_kernel(degu_p, degi_p, eu0, ei0, ru_o, ri_o, au_o, ai_o,
                  e_buf, a_buf, d0, d1, r_buf):
    c = lax.axis_index("c")
    s = lax.axis_index("s")
    w = s * NC + c
    sl = pl.ds(w * RPW, RPW)
    for deg_p, e_in, r_out, a_out in (
        (degu_p, eu0, ru_o, au_o),
        (degi_p, ei0, ri_o, ai_o),
    ):
        pltpu.sync_copy(deg_p.at[0, sl], d0)
        pltpu.sync_copy(deg_p.at[1, sl], d1)
        pltpu.sync_copy(e_in.at[sl], e_buf)

        def row_body(r, carry):
            d = d0[r, pl.ds(0, L)] + d1[r, pl.ds(0, L)]
            y = jnp.where(d > 0.0, _rsqrt16(d), 0.0)
            r_buf[r, pl.ds(0, L)] = y
            for cb in range(8):
                a_buf[r, pl.ds(cb * L, L)] = y * e_buf[r, pl.ds(cb * L, L)]
            return carry
        lax.fori_loop(0, RPW, row_body, 0)
        pltpu.sync_copy(r_buf, r_out.at[sl])
        pltpu.sync_copy(a_buf, a_out.at[sl])


@functools.partial(
    pl.kernel,
    out_type=[
        jax.ShapeDtypeStruct((NC, NT, D), _f32),
        jax.ShapeDtypeStruct((NC, NT, D), _f32),
    ],
    mesh=_mesh,
    scratch_types=[
        pltpu.VMEM_SHARED((NT, D), _f32),
        pltpu.VMEM_SHARED((NT, D), _f32),
        pltpu.VMEM((2, CH, 128), _i32),
        pltpu.VMEM((2, CH, 128), _i32),
        pltpu.VMEM((128, D), _f32),
        pltpu.VMEM((128, D), _f32),
        pltpu.SemaphoreType.DMA,
        pltpu.SemaphoreType.DMA,
        pltpu.SemaphoreType.DMA,
        pltpu.SemaphoreType.DMA,
        pltpu.SemaphoreType.DMA,
    ],
)
def _edge_kernel(u2d, i2d, au, ai, tu_o, ti_o, sh_tu, sh_ti,
                 idx_u, idx_i, buf_a, buf_b, sg0, sg1, ss0, ss1, spf):
    c = lax.axis_index("c")
    s = lax.axis_index("s")
    _fill(buf_a, 128, D // L, 0.0)
    for k0 in range(0, RPS, 80):
        pltpu.sync_copy(buf_a.at[pl.ds(0, 80)], sh_tu.at[pl.ds(s * RPS + k0, 80)])
        pltpu.sync_copy(buf_a.at[pl.ds(0, 80)], sh_ti.at[pl.ds(s * RPS + k0, 80)])
    plsc.subcore_barrier()

    r0 = c * EPC + s * EPT
    pltpu.sync_copy(u2d.at[pl.ds(r0, CH)], idx_u.at[0])
    pltpu.sync_copy(i2d.at[pl.ds(r0, CH)], idx_i.at[0])

    for ch in range(NCHK):
        slot, nxt = ch % 2, (ch + 1) % 2
        if ch > 0:
            pltpu.make_async_copy(buf_a, sh_tu.at[idx_u.at[slot, 0]], ss0).wait()
            pltpu.make_async_copy(buf_b, sh_ti.at[idx_i.at[slot, 0]], ss1).wait()
        if ch + 1 < NCHK:
            r1 = r0 + (ch + 1) * CH
            pf_u = pltpu.async_copy(u2d.at[pl.ds(r1, CH)], idx_u.at[nxt], spf)
            pf_i = pltpu.async_copy(i2d.at[pl.ds(r1, CH)], idx_i.at[nxt], spf)

        def edge_body(jj, carry, slot=slot):
            iu = idx_u.at[slot, jj]
            ii = idx_i.at[slot, jj]

            @pl.when(jj > 0)
            def _():
                pltpu.make_async_copy(buf_a, sh_tu.at[iu], ss0).wait()
            pltpu.async_copy(ai.at[ii], buf_a, sg0).wait()
            pltpu.async_copy(buf_a, sh_tu.at[iu], ss0, add=True)

            @pl.when(jj > 0)
            def _():
                pltpu.make_async_copy(buf_b, sh_ti.at[ii], ss1).wait()
            pltpu.async_copy(au.at[iu], buf_b, sg1).wait()
            pltpu.async_copy(buf_b, sh_ti.at[ii], ss1, add=True)
            return carry
        lax.fori_loop(0, CH, edge_body, 0)
        if ch + 1 < NCHK:
            pf_u.wait()
            pf_i.wait()
    pltpu.make_async_copy(buf_a, sh_tu.at[idx_u.at[(NCHK - 1) % 2, 0]], ss0).wait()
    pltpu.make_async_copy(buf_b, sh_ti.at[idx_i.at[(NCHK - 1) % 2, 0]], ss1).wait()
    plsc.subcore_barrier()

    sl = pl.ds(s * RPS, RPS)
    pltpu.sync_copy(sh_tu.at[sl], tu_o.at[c, sl])
    pltpu.sync_copy(sh_ti.at[sl], ti_o.at[c, sl])


@functools.partial(
    pl.kernel,
    out_type=[
        jax.ShapeDtypeStruct((NT, D), _f32),
        jax.ShapeDtypeStruct((NT, D), _f32),
        jax.ShapeDtypeStruct((NT, D), _f32),
        jax.ShapeDtypeStruct((NT, D), _f32),
    ],
    mesh=_mesh,
    scratch_types=[
        pltpu.VMEM((RPW, D), _f32),
        pltpu.VMEM((RPW, D), _f32),
        pltpu.VMEM((RPW, D), _f32),
        pltpu.VMEM((RPW, DEGW), _f32),
    ],
)
def _mid_kernel(ru_i, ri_i, tu_p, ti_p, eu0, ei0, au_o, ai_o, pu_o, pi_o,
                t0, t1, e_buf, r_buf):
    c = lax.axis_index("c")
    s = lax.axis_index("s")
    w = s * NC + c
    sl = pl.ds(w * RPW, RPW)
    for r_in, t_p, e_in, a_out, p_out in (
        (ru_i, tu_p, eu0, au_o, pu_o),
        (ri_i, ti_p, ei0, ai_o, pi_o),
    ):
        pltpu.sync_copy(t_p.at[0, sl], t0)
        pltpu.sync_copy(t_p.at[1, sl], t1)
        pltpu.sync_copy(e_in.at[sl], e_buf)
        pltpu.sync_copy(r_in.at[sl], r_buf)

        def row_body(r, carry):
            y = r_buf[r, pl.ds(0, L)]
            for cb in range(8):
                cs = pl.ds(cb * L, L)
                t = t0[r, cs] + t1[r, cs]
                t0[r, cs] = y * y * t
                e_buf[r, cs] = e_buf[r, cs] + y * t
            return carry
        lax.fori_loop(0, RPW, row_body, 0)
        pltpu.sync_copy(t0, a_out.at[sl])
        pltpu.sync_copy(e_buf, p_out.at[sl])


@functools.partial(
    pl.kernel,
    out_type=[
        jax.ShapeDtypeStruct((NT, D), _f32),
        jax.ShapeDtypeStruct((NT, D), _f32),
    ],
    mesh=_mesh,
    scratch_types=[
        pltpu.VMEM((RPW, D), _f32),
        pltpu.VMEM((RPW, D), _f32),
        pltpu.VMEM((RPW, D), _f32),
        pltpu.VMEM((RPW, DEGW), _f32),
    ],
)
def _final_kernel(ru_i, ri_i, tu_p, ti_p, pu_i, pi_i, su_o, si_o,
                  t0, t1, p_buf, r_buf):
    c = lax.axis_index("c")
    s = lax.axis_index("s")
    w = s * NC + c
    sl = pl.ds(w * RPW, RPW)
    for r_in, t_p, p_in, s_out in (
        (ru_i, tu_p, pu_i, su_o),
        (ri_i, ti_p, pi_i, si_o),
    ):
        pltpu.sync_copy(t_p.at[0, sl], t0)
        pltpu.sync_copy(t_p.at[1, sl], t1)
        pltpu.sync_copy(p_in.at[sl], p_buf)
        pltpu.sync_copy(r_in.at[sl], r_buf)

        def row_body(r, carry):
            y = r_buf[r, pl.ds(0, L)]
            for cb in range(8):
                cs = pl.ds(cb * L, L)
                p_buf[r, cs] = p_buf[r, cs] + y * (t0[r, cs] + t1[r, cs])
            return carry
        lax.fori_loop(0, RPW, row_body, 0)
        pltpu.sync_copy(p_buf, s_out.at[sl])


def kernel(E_u_0, E_i_0, edge_index):
    u = edge_index[:, 0].astype(_i32)
    it = edge_index[:, 1].astype(_i32) - NU
    pad = ER * 128 - NE
    padv = TRASH + (jnp.arange(pad, dtype=_i32) % (NT - NU))
    u2d = jnp.concatenate([u, padv]).reshape(ER, 128)
    i2d = jnp.concatenate([it, padv]).reshape(ER, 128)
    eu0 = jnp.pad(E_u_0, ((0, NT - NU), (0, 0)))
    ei0 = jnp.pad(E_i_0, ((0, NT - NU), (0, 0)))

    degu_p, degi_p = _deg_kernel(u2d, i2d)
    ru, ri, au1, ai1 = _scale_kernel(degu_p, degi_p, eu0, ei0)
    tu1, ti1 = _edge_kernel(u2d, i2d, au1, ai1)
    au2, ai2, pu, pi = _mid_kernel(ru, ri, tu1, ti1, eu0, ei0)
    tu2, ti2 = _edge_kernel(u2d, i2d, au2, ai2)
    su, si = _final_kernel(ru, ri, tu2, ti2, pu, pi)
    return su[:NU], si[:NU]


# speedup vs baseline: 13.8078x; 1.0825x over previous
import functools

import jax
import jax.numpy as jnp
from jax import lax
from jax.experimental import pallas as pl
from jax.experimental.pallas import tpu as pltpu
from jax.experimental.pallas import tpu_sc as plsc

NU = 5000
D = 128
NE = 320000
NC, NS, L = 2, 16, 16
NW = NC * NS
NT = 5120
TRASH = NU
ER = 2560
EPC = ER // NC
EPT = EPC // NS
CH = 16
NCHK = EPT // CH
RPW = NT // NW
RPS = NT // NS
DEGW = 128
DG = 16

_mesh = plsc.VectorSubcoreMesh(
    core_axis_name="c", subcore_axis_name="s", num_cores=NC, num_subcores=NS)

_f32 = jnp.float32
_i32 = jnp.int32


def _fill(ref, nrows, ncolblk, val, dtype=_f32):
    def body(r, carry):
        for cb in range(ncolblk):
            ref[r, pl.ds(cb * L, L)] = jnp.full((L,), val, dtype)
        return carry
    lax.fori_loop(0, nrows, body, 0)


def _rsqrt16(x):
    s = 0.5 * (x + 1.0)
    for _ in range(15):
        s = 0.5 * (s + x / s)
    return 1.0 / s


@functools.partial(
    pl.kernel,
    out_type=[
        jax.ShapeDtypeStruct((NC, NT, DG), _f32),
        jax.ShapeDtypeStruct((NC, NT, DG), _f32),
    ],
    mesh=_mesh,
    scratch_types=[
        pltpu.VMEM_SHARED((NT, DG), _f32),
        pltpu.VMEM_SHARED((NT, DG), _f32),
        pltpu.VMEM((EPT, 128), _i32),
        pltpu.VMEM((EPT, 128), _i32),
        pltpu.VMEM((128, DG), _f32),
    ],
    compiler_params=pltpu.CompilerParams(use_tc_tiling_on_sc=False),
)
def _deg_kernel(u2d, i2d, degu_o, degi_o, sh_du, sh_di, idx_u, idx_i, ones):
    c = lax.axis_index("c")
    s = lax.axis_index("s")
    _fill(ones, 128, DG // L, 0.0)
    for k0 in range(0, RPS, 80):
        pltpu.sync_copy(ones.at[pl.ds(0, 80)], sh_du.at[pl.ds(s * RPS + k0, 80)])
        pltpu.sync_copy(ones.at[pl.ds(0, 80)], sh_di.at[pl.ds(s * RPS + k0, 80)])
    _fill(ones, 128, DG // L, 1.0)
    plsc.subcore_barrier()

    r0 = c * EPC + s * EPT
    pltpu.sync_copy(u2d.at[pl.ds(r0, EPT)], idx_u)
    pltpu.sync_copy(i2d.at[pl.ds(r0, EPT)], idx_i)

    def edge_body(j, carry):
        pltpu.sync_copy(ones, sh_du.at[idx_u.at[j]], add=True)
        pltpu.sync_copy(ones, sh_di.at[idx_i.at[j]], add=True)
        return carry
    lax.fori_loop(0, EPT, edge_body, 0)
    plsc.subcore_barrier()

    sl = pl.ds(s * RPS, RPS)
    pltpu.sync_copy(sh_du.at[sl], degu_o.at[c, sl])
    pltpu.sync_copy(sh_di.at[sl], degi_o.at[c, sl])


@functools.partial(
    pl.kernel,
    out_type=[
        jax.ShapeDtypeStruct((NT, DEGW), _f32),
        jax.ShapeDtypeStruct((NT, DEGW), _f32),
        jax.ShapeDtypeStruct((NT, D), _f32),
        jax.ShapeDtypeStruct((NT, D), _f32),
    ],
    mesh=_mesh,
    scratch_types=[
        pltpu.VMEM((RPW, D), _f32),
        pltpu.VMEM((RPW, D), _f32),
        pltpu.VMEM((RPW, DG), _f32),
        pltpu.VMEM((RPW, DG), _f32),
        pltpu.VMEM((RPW, DEGW), _f32),
    ],
    compiler_params=pltpu.CompilerParams(use_tc_tiling_on_sc=False),
)
def _scale_kernel(degu_p, degi_p, eu0, ei0, ru_o, ri_o, au_o, ai_o,
                  e_buf, a_buf, d0, d1, r_buf):
    c = lax.axis_index("c")
    s = lax.axis_index("s")
    w = s * NC + c
    sl = pl.ds(w * RPW, RPW)
    for deg_p, e_in, r_out, a_out in (
        (degu_p, eu0, ru_o, au_o),
        (degi_p, ei0, ri_o, ai_o),
    ):
        pltpu.sync_copy(deg_p.at[0, sl], d0)
        pltpu.sync_copy(deg_p.at[1, sl], d1)
        pltpu.sync_copy(e_in.at[sl], e_buf)

        def row_body(r, carry):
            d = d0[r, pl.ds(0, L)] + d1[r, pl.ds(0, L)]
            y = jnp.where(d > 0.0, _rsqrt16(d), 0.0)
            r_buf[r, pl.ds(0, L)] = y
            for cb in range(8):
                a_buf[r, pl.ds(cb * L, L)] = y * e_buf[r, pl.ds(cb * L, L)]
            return carry
        lax.fori_loop(0, RPW, row_body, 0)
        pltpu.sync_copy(r_buf, r_out.at[sl])
        pltpu.sync_copy(a_buf, a_out.at[sl])


@functools.partial(
    pl.kernel,
    out_type=[
        jax.ShapeDtypeStruct((NC, NT, D), _f32),
        jax.ShapeDtypeStruct((NC, NT, D), _f32),
    ],
    mesh=_mesh,
    scratch_types=[
        pltpu.VMEM_SHARED((NT, D), _f32),
        pltpu.VMEM_SHARED((NT, D), _f32),
        pltpu.VMEM((2, CH, 128), _i32),
        pltpu.VMEM((2, CH, 128), _i32),
        pltpu.VMEM((128, D), _f32),
        pltpu.VMEM((128, D), _f32),
        pltpu.SemaphoreType.DMA,
        pltpu.SemaphoreType.DMA,
        pltpu.SemaphoreType.DMA,
        pltpu.SemaphoreType.DMA,
        pltpu.SemaphoreType.DMA,
    ],
)
def _edge_kernel(u2d, i2d, au, ai, tu_o, ti_o, sh_tu, sh_ti,
                 idx_u, idx_i, buf_a, buf_b, sg0, sg1, ss0, ss1, spf):
    c = lax.axis_index("c")
    s = lax.axis_index("s")
    _fill(buf_a, 128, D // L, 0.0)
    for k0 in range(0, RPS, 80):
        pltpu.sync_copy(buf_a.at[pl.ds(0, 80)], sh_tu.at[pl.ds(s * RPS + k0, 80)])
        pltpu.sync_copy(buf_a.at[pl.ds(0, 80)], sh_ti.at[pl.ds(s * RPS + k0, 80)])
    plsc.subcore_barrier()

    r0 = c * EPC + s * EPT
    pltpu.sync_copy(u2d.at[pl.ds(r0, CH)], idx_u.at[0])
    pltpu.sync_copy(i2d.at[pl.ds(r0, CH)], idx_i.at[0])

    for ch in range(NCHK):
        slot, nxt = ch % 2, (ch + 1) % 2
        if ch > 0:
            pltpu.make_async_copy(buf_a, sh_tu.at[idx_u.at[slot, 0]], ss0).wait()
            pltpu.make_async_copy(buf_b, sh_ti.at[idx_i.at[slot, 0]], ss1).wait()
        if ch + 1 < NCHK:
            r1 = r0 + (ch + 1) * CH
            pf_u = pltpu.async_copy(u2d.at[pl.ds(r1, CH)], idx_u.at[nxt], spf)
            pf_i = pltpu.async_copy(i2d.at[pl.ds(r1, CH)], idx_i.at[nxt], spf)

        def edge_body(jj, carry, slot=slot):
            iu = idx_u.at[slot, jj]
            ii = idx_i.at[slot, jj]

            @pl.when(jj > 0)
            def _():
                pltpu.make_async_copy(buf_a, sh_tu.at[iu], ss0).wait()
            pltpu.async_copy(ai.at[ii], buf_a, sg0).wait()
            pltpu.async_copy(buf_a, sh_tu.at[iu], ss0, add=True)

            @pl.when(jj > 0)
            def _():
                pltpu.make_async_copy(buf_b, sh_ti.at[ii], ss1).wait()
            pltpu.async_copy(au.at[iu], buf_b, sg1).wait()
            pltpu.async_copy(buf_b, sh_ti.at[ii], ss1, add=True)
            return carry
        lax.fori_loop(0, CH, edge_body, 0)
        if ch + 1 < NCHK:
            pf_u.wait()
            pf_i.wait()
    pltpu.make_async_copy(buf_a, sh_tu.at[idx_u.at[(NCHK - 1) % 2, 0]], ss0).wait()
    pltpu.make_async_copy(buf_b, sh_ti.at[idx_i.at[(NCHK - 1) % 2, 0]], ss1).wait()
    plsc.subcore_barrier()

    sl = pl.ds(s * RPS, RPS)
    pltpu.sync_copy(sh_tu.at[sl], tu_o.at[c, sl])
    pltpu.sync_copy(sh_ti.at[sl], ti_o.at[c, sl])


@functools.partial(
    pl.kernel,
    out_type=[
        jax.ShapeDtypeStruct((NT, D), _f32),
        jax.ShapeDtypeStruct((NT, D), _f32),
        jax.ShapeDtypeStruct((NT, D), _f32),
        jax.ShapeDtypeStruct((NT, D), _f32),
    ],
    mesh=_mesh,
    scratch_types=[
        pltpu.VMEM((RPW, D), _f32),
        pltpu.VMEM((RPW, D), _f32),
        pltpu.VMEM((RPW, D), _f32),
        pltpu.VMEM((RPW, DEGW), _f32),
    ],
)
def _mid_kernel(ru_i, ri_i, tu_p, ti_p, eu0, ei0, au_o, ai_o, pu_o, pi_o,
                t0, t1, e_buf, r_buf):
    c = lax.axis_index("c")
    s = lax.axis_index("s")
    w = s * NC + c
    sl = pl.ds(w * RPW, RPW)
    for r_in, t_p, e_in, a_out, p_out in (
        (ru_i, tu_p, eu0, au_o, pu_o),
        (ri_i, ti_p, ei0, ai_o, pi_o),
    ):
        pltpu.sync_copy(t_p.at[0, sl], t0)
        pltpu.sync_copy(t_p.at[1, sl], t1)
        pltpu.sync_copy(e_in.at[sl], e_buf)
        pltpu.sync_copy(r_in.at[sl], r_buf)

        def row_body(r, carry):
            y = r_buf[r, pl.ds(0, L)]
            for cb in range(8):
                cs = pl.ds(cb * L, L)
                t = t0[r, cs] + t1[r, cs]
                t0[r, cs] = y * y * t
                e_buf[r, cs] = e_buf[r, cs] + y * t
            return carry
        lax.fori_loop(0, RPW, row_body, 0)
        pltpu.sync_copy(t0, a_out.at[sl])
        pltpu.sync_copy(e_buf, p_out.at[sl])


@functools.partial(
    pl.kernel,
    out_type=[
        jax.ShapeDtypeStruct((NT, D), _f32),
        jax.ShapeDtypeStruct((NT, D), _f32),
    ],
    mesh=_mesh,
    scratch_types=[
        pltpu.VMEM((RPW, D), _f32),
        pltpu.VMEM((RPW, D), _f32),
        pltpu.VMEM((RPW, D), _f32),
        pltpu.VMEM((RPW, DEGW), _f32),
    ],
)
def _final_kernel(ru_i, ri_i, tu_p, ti_p, pu_i, pi_i, su_o, si_o,
                  t0, t1, p_buf, r_buf):
    c = lax.axis_index("c")
    s = lax.axis_index("s")
    w = s * NC + c
    sl = pl.ds(w * RPW, RPW)
    for r_in, t_p, p_in, s_out in (
        (ru_i, tu_p, pu_i, su_o),
        (ri_i, ti_p, pi_i, si_o),
    ):
        pltpu.sync_copy(t_p.at[0, sl], t0)
        pltpu.sync_copy(t_p.at[1, sl], t1)
        pltpu.sync_copy(p_in.at[sl], p_buf)
        pltpu.sync_copy(r_in.at[sl], r_buf)

        def row_body(r, carry):
            y = r_buf[r, pl.ds(0, L)]
            for cb in range(8):
                cs = pl.ds(cb * L, L)
                p_buf[r, cs] = p_buf[r, cs] + y * (t0[r, cs] + t1[r, cs])
            return carry
        lax.fori_loop(0, RPW, row_body, 0)
        pltpu.sync_copy(p_buf, s_out.at[sl])


def kernel(E_u_0, E_i_0, edge_index):
    u = edge_index[:, 0].astype(_i32)
    it = edge_index[:, 1].astype(_i32) - NU
    pad = ER * 128 - NE
    padv = TRASH + (jnp.arange(pad, dtype=_i32) % (NT - NU))
    u2d = jnp.concatenate([u, padv]).reshape(ER, 128)
    i2d = jnp.concatenate([it, padv]).reshape(ER, 128)
    eu0 = jnp.pad(E_u_0, ((0, NT - NU), (0, 0)))
    ei0 = jnp.pad(E_i_0, ((0, NT - NU), (0, 0)))

    degu_p, degi_p = _deg_kernel(u2d, i2d)
    ru, ri, au1, ai1 = _scale_kernel(degu_p, degi_p, eu0, ei0)
    tu1, ti1 = _edge_kernel(u2d, i2d, au1, ai1)
    au2, ai2, pu, pi = _mid_kernel(ru, ri, tu1, ti1, eu0, ei0)
    tu2, ti2 = _edge_kernel(u2d, i2d, au2, ai2)
    su, si = _final_kernel(ru, ri, tu2, ti2, pu, pi)
    return su[:NU], si[:NU]
